# baseline (speedup 1.0000x reference)
import jax
import jax.numpy as jnp
from jax import lax
from jax.experimental import pallas as pl
from jax.experimental.pallas import tpu as pltpu
from jax.experimental.pallas import tpu_sc as plsc

_NRES = 4096
_NAA = 21
_NC = 2
_NS = 16
_L = 16
_CH = 64
_RTOT = 2 * _NRES
_RPT = _RTOT // _NS
_BLK = 2048
_W = 128


def _sc_body(xr, idxf, g8, b8, sums_o, x_v, i_v, g_v, b_v, acc_s):
    cid = lax.axis_index("c")
    sid = lax.axis_index("s")
    wid = sid * _NC + cid
    n_atoms = idxf.shape[0]
    t_per_w = n_atoms // (_NC * _NS) // _CH

    pltpu.sync_copy(g8, g_v)
    pltpu.sync_copy(b8, b_v)

    zv = jnp.zeros((_L,), jnp.float32)

    def zrow(i, carry):
        for k in range(8):
            x_v[i, k * _L:(k + 1) * _L] = zv
        return carry

    lax.fori_loop(0, _CH, zrow, 0)

    def zcp(i, carry):
        pltpu.sync_copy(x_v, acc_s.at[pl.ds(sid * _RPT + i * _CH, _CH)])
        return carry

    lax.fori_loop(0, _RPT // _CH, zcp, 0)
    plsc.subcore_barrier()

    gs = [g_v[k] for k in range(8)]
    bs = [b_v[k] for k in range(8)]
    lane = lax.broadcasted_iota(jnp.int32, (_L,), 0)
    perms = [lane ^ sh for sh in (8, 4, 2, 1)]

    def lanesum(v):
        for p in perms:
            v = v + jnp.take(v, p)
        return v

    def chunk(t, carry):
        start = (wid * t_per_w + t) * _CH
        pltpu.sync_copy(xr.at[pl.ds(start, _CH)], x_v)
        pltpu.sync_copy(idxf.at[pl.ds(start, _CH)], i_v)

        def atom(a, c2):
            vs = [x_v[a, k * _L:(k + 1) * _L] for k in range(8)]
            s = vs[0]
            for k in range(1, 8):
                s = s + vs[k]
            mu_v = lanesum(s) * 0.0078125
            ds_ = [v - mu_v for v in vs]
            ss = ds_[0] * ds_[0]
            for k in range(1, 8):
                ss = ss + ds_[k] * ds_[k]
            vv = lanesum(ss) * 0.0078125 + 1e-5
            iv = plsc.bitcast(vv, jnp.int32)
            iv = jnp.int32(0x5F3759DF) - (iv >> 1)
            yv = plsc.bitcast(iv, jnp.float32)
            for _ in range(3):
                yv = yv * (1.5 - 0.5 * vv * yv * yv)
            for k in range(8):
                x_v[a, k * _L:(k + 1) * _L] = ds_[k] * yv * gs[k] + bs[k]
            return c2

        lax.fori_loop(0, _CH, atom, 0, unroll=2)
        pltpu.sync_copy(x_v, acc_s.at[i_v], add=True)
        return carry

    lax.fori_loop(0, t_per_w, chunk, 0)
    plsc.subcore_barrier()

    def dcp(i, carry):
        pltpu.sync_copy(acc_s.at[pl.ds(sid * _RPT + i * _CH, _CH)], x_v)
        pltpu.sync_copy(
            x_v, sums_o.at[pl.ds(cid * _RTOT + sid * _RPT + i * _CH, _CH)])
        return carry

    lax.fori_loop(0, _RPT // _CH, dcp, 0)


def _epi_body(idx_ref, sp_ref, wpt_ref, wot_ref, bo_ref, out_ref, cnt_ref):
    j = pl.program_id(1)
    nblk = pl.num_programs(1)

    @pl.when(j == 0)
    def _init():
        cnt_ref[...] = jnp.zeros_like(cnt_ref)

    idx = idx_ref[0, 0]
    r_first = idx_ref[0, 0, 0, 0]
    r_last = idx_ref[0, 0, 0, _BLK - 1]
    w0 = r_first // _W
    nwin = r_last // _W - w0 + 1

    def body(w, carry):
        base = (w0 + w) * _W
        cols = base + jax.lax.broadcasted_iota(jnp.int32, (_W, 1), 0)
        mask = cols == idx
        cnt = jnp.sum(mask.astype(jnp.float32), axis=1, keepdims=True)
        cnt_ref[pl.ds(base, _W), :] += cnt
        return carry

    jax.lax.fori_loop(0, nwin, body, 0)

    @pl.when(j == nblk - 1)
    def _epilogue():
        s = sp_ref[0, 0] + sp_ref[1, 0]
        t = jax.lax.dot(s, wpt_ref[...], preferred_element_type=jnp.float32)
        t = t / (cnt_ref[...] + 1.0)
        o = jax.lax.dot(t, wot_ref[...], preferred_element_type=jnp.float32)
        out_ref[0] = o + bo_ref[...]


def kernel(atom_embed, atom_res_idx, fastpass, ln_gamma, ln_beta,
           W_proj, W_out, b_out):
    del fastpass
    b, n, c = atom_embed.shape
    xr = atom_embed.reshape(b * n, c)
    idx32 = atom_res_idx.astype(jnp.int32)
    idxf = (idx32 + (jnp.arange(b, dtype=jnp.int32) * _NRES)[:, None]
            ).reshape(-1)
    g8 = ln_gamma.astype(jnp.float32).reshape(8, _L)
    b8 = ln_beta.astype(jnp.float32).reshape(8, _L)

    mesh = plsc.VectorSubcoreMesh(core_axis_name="c", subcore_axis_name="s")
    sums_p = pl.kernel(
        _sc_body,
        mesh=mesh,
        out_type=jax.ShapeDtypeStruct((_NC * _RTOT, c), jnp.float32),
        scratch_types=[
            pltpu.VMEM((_CH, c), jnp.float32),
            pltpu.VMEM((_CH,), jnp.int32),
            pltpu.VMEM((8, _L), jnp.float32),
            pltpu.VMEM((8, _L), jnp.float32),
            pltpu.VMEM_SHARED((_RTOT, c), jnp.float32),
        ],
        compiler_params=pltpu.CompilerParams(needs_layout_passes=False),
    )(xr, idxf, g8, b8)

    nblk = n // _BLK
    idxr = idx32.reshape(b, nblk, 1, _BLK)
    wpt = W_proj.T.astype(jnp.float32)
    wot = jnp.zeros((c, c), jnp.float32).at[:, :_NAA].set(W_out.T)
    bo = jnp.zeros((1, c), jnp.float32).at[0, :_NAA].set(b_out)

    out = pl.pallas_call(
        _epi_body,
        grid=(b, nblk),
        in_specs=[
            pl.BlockSpec((1, 1, 1, _BLK), lambda bi, ji: (bi, ji, 0, 0)),
            pl.BlockSpec((_NC, 1, _NRES, c), lambda bi, ji: (0, bi, 0, 0)),
            pl.BlockSpec((c, c), lambda bi, ji: (0, 0)),
            pl.BlockSpec((c, c), lambda bi, ji: (0, 0)),
            pl.BlockSpec((1, c), lambda bi, ji: (0, 0)),
        ],
        out_specs=pl.BlockSpec((1, _NRES, c), lambda bi, ji: (bi, 0, 0)),
        out_shape=jax.ShapeDtypeStruct((b, _NRES, c), jnp.float32),
        scratch_shapes=[
            pltpu.VMEM((_NRES, 1), jnp.float32),
        ],
        compiler_params=pltpu.CompilerParams(
            dimension_semantics=("arbitrary", "arbitrary")),
    )(idxr, sums_p.reshape(_NC, b, _NRES, c), wpt, wot, bo)
    return out[..., :_NAA]


# speedup vs baseline: 2.3244x; 2.3244x over previous
import jax
import jax.numpy as jnp
from jax import lax
from jax.experimental import pallas as pl
from jax.experimental.pallas import tpu as pltpu
from jax.experimental.pallas import tpu_sc as plsc

_NRES = 4096
_NAA = 21
_NC = 2
_NS = 16
_L = 16
_CH = 64
_RTOT = 2 * _NRES
_RPT = _RTOT // _NS
_BLK = 2048
_W = 128
_NA_SC = 65536


def _sc_body(xr, idxf, g8, b8, sums_o, x_v, i_v, g_v, b_v, acc_s):
    cid = lax.axis_index("c")
    sid = lax.axis_index("s")
    wid = sid * _NC + cid
    na_tc = idxf.shape[0] - _NA_SC
    t_per_w = _NA_SC // (_NC * _NS) // _CH

    pltpu.sync_copy(g8, g_v)
    pltpu.sync_copy(b8, b_v)

    zv = jnp.zeros((_L,), jnp.float32)

    def zrow(i, carry):
        for k in range(8):
            x_v[i, k * _L:(k + 1) * _L] = zv
        return carry

    lax.fori_loop(0, _CH, zrow, 0)

    def zcp(i, carry):
        pltpu.sync_copy(x_v, acc_s.at[pl.ds(sid * _RPT + i * _CH, _CH)])
        return carry

    lax.fori_loop(0, _RPT // _CH, zcp, 0)
    plsc.subcore_barrier()

    gs = [g_v[k] for k in range(8)]
    bs = [b_v[k] for k in range(8)]
    lane = lax.broadcasted_iota(jnp.int32, (_L,), 0)
    perms = [lane ^ sh for sh in (8, 4, 2, 1)]

    def lanesum(v):
        for p in perms:
            v = v + jnp.take(v, p)
        return v

    def chunk(t, carry):
        start = na_tc + (wid * t_per_w + t) * _CH
        pltpu.sync_copy(xr.at[pl.ds(start, _CH)], x_v)
        pltpu.sync_copy(idxf.at[pl.ds(start, _CH)], i_v)

        def atom(a, c2):
            vs = [x_v[a, k * _L:(k + 1) * _L] for k in range(8)]
            s = vs[0]
            for k in range(1, 8):
                s = s + vs[k]
            mu_v = lanesum(s) * 0.0078125
            ds_ = [v - mu_v for v in vs]
            ss = ds_[0] * ds_[0]
            for k in range(1, 8):
                ss = ss + ds_[k] * ds_[k]
            vv = lanesum(ss) * 0.0078125 + 1e-5
            iv = plsc.bitcast(vv, jnp.int32)
            iv = jnp.int32(0x5F3759DF) - (iv >> 1)
            yv = plsc.bitcast(iv, jnp.float32)
            for _ in range(3):
                yv = yv * (1.5 - 0.5 * vv * yv * yv)
            for k in range(8):
                x_v[a, k * _L:(k + 1) * _L] = ds_[k] * yv * gs[k] + bs[k]
            return c2

        lax.fori_loop(0, _CH, atom, 0, unroll=2)
        pltpu.sync_copy(x_v, acc_s.at[i_v], add=True)
        return carry

    lax.fori_loop(0, t_per_w, chunk, 0)
    plsc.subcore_barrier()

    def dcp(i, carry):
        pltpu.sync_copy(acc_s.at[pl.ds(sid * _RPT + i * _CH, _CH)], x_v)
        pltpu.sync_copy(
            x_v, sums_o.at[pl.ds(cid * _RTOT + sid * _RPT + i * _CH, _CH)])
        return carry

    lax.fori_loop(0, _RPT // _CH, dcp, 0)


def _tc_body(idx_ref, x_ref, g_ref, bta_ref, sums_ref, acc_ref):
    j = pl.program_id(0)
    nblk = pl.num_programs(0)

    @pl.when(j == 0)
    def _init():
        acc_ref[...] = jnp.zeros_like(acc_ref)

    x = x_ref[...]
    mu = jnp.mean(x, axis=-1, keepdims=True)
    xc = x - mu
    var = jnp.mean(xc * xc, axis=-1, keepdims=True)
    y = xc * jax.lax.rsqrt(var + 1e-5) * g_ref[...] + bta_ref[...]
    yb = y.astype(jnp.bfloat16)

    idx = idx_ref[0]
    r_first = idx_ref[0, 0, 0]
    r_last = idx_ref[0, 0, _BLK - 1]
    w0 = r_first // _W
    nwin = r_last // _W - w0 + 1

    def body(w, carry):
        base = (w0 + w) * _W
        cols = base + jax.lax.broadcasted_iota(jnp.int32, (_W, 1), 0)
        oh = (cols == idx).astype(jnp.bfloat16)
        part = jax.lax.dot(oh, yb, preferred_element_type=jnp.float32)
        acc_ref[pl.ds(base, _W), :] += part
        return carry

    jax.lax.fori_loop(0, nwin, body, 0)

    @pl.when(j == nblk - 1)
    def _dump():
        sums_ref[...] = acc_ref[...]


def _epi_body(idx_ref, sp_ref, wpt_ref, wot_ref, bo_ref, out_ref, cnt_ref):
    j = pl.program_id(1)
    nblk = pl.num_programs(1)

    @pl.when(j == 0)
    def _init():
        cnt_ref[...] = jnp.zeros_like(cnt_ref)

    idx = idx_ref[0, 0]
    r_first = idx_ref[0, 0, 0, 0]
    r_last = idx_ref[0, 0, 0, _BLK - 1]
    w0 = r_first // _W
    nwin = r_last // _W - w0 + 1

    def body(w, carry):
        base = (w0 + w) * _W
        cols = base + jax.lax.broadcasted_iota(jnp.int32, (_W, 1), 0)
        mask = cols == idx
        cnt = jnp.sum(mask.astype(jnp.float32), axis=1, keepdims=True)
        cnt_ref[pl.ds(base, _W), :] += cnt
        return carry

    jax.lax.fori_loop(0, nwin, body, 0)

    @pl.when(j == nblk - 1)
    def _epilogue():
        s = sp_ref[0, 0] + sp_ref[1, 0] + sp_ref[2, 0]
        t = jax.lax.dot(s, wpt_ref[...], preferred_element_type=jnp.float32)
        t = t / (cnt_ref[...] + 1.0)
        o = jax.lax.dot(t, wot_ref[...], preferred_element_type=jnp.float32)
        out_ref[0] = o + bo_ref[...]


def kernel(atom_embed, atom_res_idx, fastpass, ln_gamma, ln_beta,
           W_proj, W_out, b_out):
    del fastpass
    b, n, c = atom_embed.shape
    xr = atom_embed.reshape(b * n, c)
    idx32 = atom_res_idx.astype(jnp.int32)
    idxf = (idx32 + (jnp.arange(b, dtype=jnp.int32) * _NRES)[:, None]
            ).reshape(-1)
    g8 = ln_gamma.astype(jnp.float32).reshape(8, _L)
    b8 = ln_beta.astype(jnp.float32).reshape(8, _L)

    na_tc = b * n - _NA_SC
    mesh = plsc.VectorSubcoreMesh(core_axis_name="c", subcore_axis_name="s")
    sums_sc = pl.kernel(
        _sc_body,
        mesh=mesh,
        out_type=jax.ShapeDtypeStruct((_NC * _RTOT, c), jnp.float32),
        scratch_types=[
            pltpu.VMEM((_CH, c), jnp.float32),
            pltpu.VMEM((_CH,), jnp.int32),
            pltpu.VMEM((8, _L), jnp.float32),
            pltpu.VMEM((8, _L), jnp.float32),
            pltpu.VMEM_SHARED((_RTOT, c), jnp.float32),
        ],
        compiler_params=pltpu.CompilerParams(needs_layout_passes=False),
    )(xr, idxf, g8, b8)

    g = ln_gamma.reshape(1, c).astype(jnp.float32)
    bta = ln_beta.reshape(1, c).astype(jnp.float32)
    nblk_tc = na_tc // _BLK
    idxr_tc = idxf.reshape(b * n // _BLK, 1, _BLK)
    sums_tc = pl.pallas_call(
        _tc_body,
        grid=(nblk_tc,),
        in_specs=[
            pl.BlockSpec((1, 1, _BLK), lambda ji: (ji, 0, 0)),
            pl.BlockSpec((_BLK, c), lambda ji: (ji, 0)),
            pl.BlockSpec((1, c), lambda ji: (0, 0)),
            pl.BlockSpec((1, c), lambda ji: (0, 0)),
        ],
        out_specs=pl.BlockSpec((_RTOT, c), lambda ji: (0, 0)),
        out_shape=jax.ShapeDtypeStruct((_RTOT, c), jnp.float32),
        scratch_shapes=[
            pltpu.VMEM((_RTOT, c), jnp.float32),
        ],
        compiler_params=pltpu.CompilerParams(
            dimension_semantics=("arbitrary",)),
    )(idxr_tc, xr, g, bta)

    sums_p = jnp.concatenate([sums_sc, sums_tc], axis=0)

    nblk = n // _BLK
    idxr = idx32.reshape(b, nblk, 1, _BLK)
    wpt = W_proj.T.astype(jnp.float32)
    wot = jnp.zeros((c, c), jnp.float32).at[:, :_NAA].set(W_out.T)
    bo = jnp.zeros((1, c), jnp.float32).at[0, :_NAA].set(b_out)

    out = pl.pallas_call(
        _epi_body,
        grid=(b, nblk),
        in_specs=[
            pl.BlockSpec((1, 1, 1, _BLK), lambda bi, ji: (bi, ji, 0, 0)),
            pl.BlockSpec((_NC + 1, 1, _NRES, c), lambda bi, ji: (0, bi, 0, 0)),
            pl.BlockSpec((c, c), lambda bi, ji: (0, 0)),
            pl.BlockSpec((c, c), lambda bi, ji: (0, 0)),
            pl.BlockSpec((1, c), lambda bi, ji: (0, 0)),
        ],
        out_specs=pl.BlockSpec((1, _NRES, c), lambda bi, ji: (bi, 0, 0)),
        out_shape=jax.ShapeDtypeStruct((b, _NRES, c), jnp.float32),
        scratch_shapes=[
            pltpu.VMEM((_NRES, 1), jnp.float32),
        ],
        compiler_params=pltpu.CompilerParams(
            dimension_semantics=("arbitrary", "arbitrary")),
    )(idxr, sums_p.reshape(_NC + 1, b, _NRES, c), wpt, wot, bo)
    return out[..., :_NAA]
